# Initial kernel scaffold; baseline (speedup 1.0000x reference)
import jax
import jax.numpy as jnp
from jax.experimental import pallas as pl


def kernel(source, target, encoder_outputs, encoder_hidden, embed, W_ih, W_hh, b_ih, W_out, b_out):
    raise NotImplementedError("write your pallas kernel here")


# speedup vs baseline: 5.6644x; 5.6644x over previous
import functools

import jax
import jax.numpy as jnp
from jax.experimental import pallas as pl
from jax.experimental.pallas import tpu as pltpu

_K = 8
_SOS = 2
_EOS = 3
_NEG = -1.0e30
_REMOVED = -3.0e30


def _topk8(vals, ids):
    out_v = []
    out_i = []
    big = jnp.int32(2**30)
    work = vals
    for _ in range(_K):
        m = jnp.max(work, axis=1, keepdims=True)
        ismax = work == m
        cid = jnp.min(jnp.where(ismax, ids, big), axis=1, keepdims=True)
        out_v.append(m)
        out_i.append(cid)
        work = jnp.where(ids == cid, _REMOVED, work)
    return jnp.concatenate(out_v, axis=1), jnp.concatenate(out_i, axis=1)


def _decode_body(B, V, H, T,
                 h0_ref, embed_ref, wih_ref, whh_ref, bih_ref, wout_ref,
                 bout_ref, outs_ref, seqs_ref, scores_ref, logsm_store):
    f32 = jnp.float32
    R = _K * B

    h = h0_ref[...]
    row = jax.lax.broadcasted_iota(jnp.int32, (R, 1), 0)
    bcol = row % B
    seq = jnp.where(row < B, 0.0, -jnp.inf).astype(f32)
    inp = jnp.full((R, 1), _SOS, jnp.int32)
    cols_v = jax.lax.broadcasted_iota(jnp.int32, (R, V), 1)
    cols_r = jax.lax.broadcasted_iota(jnp.int32, (R, R), 1)

    sym_hist = []
    pred_hist = []

    for t in range(T):
        onehot_e = (inp == cols_v).astype(f32)
        x = jnp.dot(onehot_e, embed_ref[...], preferred_element_type=f32)
        gi = jnp.dot(x, wih_ref[...], preferred_element_type=f32) + bih_ref[...]
        gh = jnp.dot(h, whh_ref[...], preferred_element_type=f32)
        r = jax.nn.sigmoid(gi[:, 0:H] + gh[:, 0:H])
        z = jax.nn.sigmoid(gi[:, H:2 * H] + gh[:, H:2 * H])
        n = jnp.tanh(gi[:, 2 * H:3 * H] + r * gh[:, 2 * H:3 * H])
        hs = (1.0 - z) * n + z * h

        logits = jnp.dot(hs, wout_ref[...], preferred_element_type=f32) + bout_ref[...]
        m = jnp.max(logits, axis=1, keepdims=True)
        lse = m + jnp.log(jnp.sum(jnp.exp(logits - m), axis=1, keepdims=True))
        log_sm = logits - lse
        logsm_store[t] = log_sm

        infl = jnp.maximum(log_sm + seq, _NEG)
        v1, c1 = _topk8(infl, cols_v)

        v2 = jnp.concatenate([v1[k * B:(k + 1) * B, :] for k in range(_K)], axis=1)
        c2 = jnp.concatenate([c1[k * B:(k + 1) * B, :] + k * V for k in range(_K)],
                             axis=1)
        scores_t, cand_t = _topk8(v2, c2)

        sym = cand_t % V
        pred = cand_t // V
        sym_hist.append(sym)
        pred_hist.append(pred)

        if t == T - 1:
            scores_ref[...] = jnp.where(scores_t <= 0.9 * _NEG, -jnp.inf, scores_t)

        inp = jnp.concatenate([sym[:, k:k + 1] for k in range(_K)], axis=0)
        pred_col = jnp.concatenate([pred[:, k:k + 1] for k in range(_K)], axis=0)
        seq_new = jnp.concatenate(
            [scores_t[:, k:k + 1] for k in range(_K)], axis=0)
        seq = jnp.where(inp == _EOS, -jnp.inf, seq_new)

        prow = pred_col * B + bcol
        onehot_h = (prow == cols_r).astype(f32)
        h = jnp.dot(onehot_h, hs, preferred_element_type=f32)

    ptr = jax.lax.broadcasted_iota(jnp.int32, (B, _K), 1)
    for t in range(T - 1, -1, -1):
        sym_t = sym_hist[t]
        pred_t = pred_hist[t]
        s = jnp.zeros((B, _K), jnp.int32)
        p = jnp.zeros((B, _K), jnp.int32)
        for j in range(_K):
            sel = ptr == j
            s = jnp.where(sel, sym_t[:, j:j + 1], s)
            p = jnp.where(sel, pred_t[:, j:j + 1], p)
        seqs_ref[t] = s
        ptr0 = ptr[:, 0:1]
        ls = logsm_store[t]
        o = jnp.zeros((B, V), f32)
        for k in range(_K):
            o = o + jnp.where(ptr0 == k, 1.0, 0.0) * ls[k * B:(k + 1) * B, :]
        outs_ref[t] = o
        ptr = p


def kernel(source, target, encoder_outputs, encoder_hidden, embed, W_ih,
           W_hh, b_ih, W_out, b_out):
    B = source.shape[1]
    T = target.shape[0]
    V, H = embed.shape
    f32 = jnp.float32

    init_idx = ((jnp.arange(B)[None, :] * _K + jnp.arange(_K)[:, None]) % B
                ).reshape(-1)
    h0 = encoder_hidden[0][init_idx]

    body = functools.partial(_decode_body, B, V, H, T)
    outs, seqs, scores = pl.pallas_call(
        body,
        out_shape=[
            jax.ShapeDtypeStruct((T, B, V), f32),
            jax.ShapeDtypeStruct((T, B, _K), jnp.int32),
            jax.ShapeDtypeStruct((B, _K), f32),
        ],
        scratch_shapes=[pltpu.VMEM((T, _K * B, V), f32)],
        compiler_params=pltpu.CompilerParams(
            vmem_limit_bytes=128 * 1024 * 1024),
    )(h0, embed, W_ih, W_hh, b_ih.reshape(1, 3 * H), W_out,
      b_out.reshape(1, V))
    return outs, seqs, scores
